# baseline (speedup 1.0000x reference)
import functools

import jax
import jax.numpy as jnp
from jax import lax
from jax.experimental import pallas as pl
from jax.experimental.pallas import tpu as pltpu
from jax.experimental.pallas import tpu_sc as plsc

N_SC = 2
N_TILE = 16
NW = N_SC * N_TILE
B_EDGE = 128
DEG_W = 8


def _sc_mesh():
    return plsc.VectorSubcoreMesh(core_axis_name="c", subcore_axis_name="s")


def _sc_segsum(g, srcp, dstp, zrows, nacc, rpt):
    n_nodes, d = g.shape
    nb = srcp.shape[1]

    @functools.partial(
        pl.kernel,
        mesh=_sc_mesh(),
        compiler_params=pltpu.CompilerParams(use_tc_tiling_on_sc=False),
        out_type=jax.ShapeDtypeStruct((N_SC, nacc, d), jnp.float32),
        scratch_types=[
            pltpu.VMEM((nb, B_EDGE), jnp.int32),
            pltpu.VMEM((nb, B_EDGE), jnp.int32),
            [pltpu.VMEM((B_EDGE, d), jnp.float32)] * 8,
            pltpu.VMEM_SHARED((nacc, d), jnp.float32),
            [pltpu.SemaphoreType.DMA] * 8,
            [pltpu.SemaphoreType.DMA] * 8,
        ],
    )
    def seg(g_hbm, src_hbm, dst_hbm, z_hbm, out_hbm,
            src_v, dst_v, bufs, acc, gsems, ssems):
        cid = lax.axis_index("c")
        sid = lax.axis_index("s")
        wid = cid * N_TILE + sid

        pltpu.sync_copy(src_hbm.at[wid], src_v)
        pltpu.sync_copy(dst_hbm.at[wid], dst_v)
        pltpu.sync_copy(z_hbm, acc.at[pl.ds(sid * rpt, rpt)])
        plsc.subcore_barrier()

        def gather_start(j, b):
            pltpu.make_async_copy(g_hbm.at[src_v.at[j]], bufs[b], gsems[b]).start()

        def gather_wait(b):
            pltpu.make_async_copy(g_hbm.at[src_v.at[0]], bufs[b], gsems[b]).wait()

        def scatter_start(j, b):
            pltpu.make_async_copy(bufs[b], acc.at[dst_v.at[j]], ssems[b]).start(add=True)

        def scatter_wait(b):
            pltpu.make_async_copy(bufs[b], acc.at[dst_v.at[0]], ssems[b]).wait()

        nbuf = 8
        look = nbuf // 2
        for k in range(look):
            gather_start(k, k)

        def body(i, _):
            for b in range(nbuf):
                j = nbuf * i + b
                gather_wait(b)
                scatter_start(j, b)
                nxt = (b + look) % nbuf

                @pl.when(j + look < nb)
                def _():
                    @pl.when(j >= look)
                    def _():
                        scatter_wait(nxt)

                    gather_start(j + look, nxt)
            return 0

        lax.fori_loop(0, nb // nbuf, body, 0)
        for k in range(nbuf):
            scatter_wait(k)
        plsc.subcore_barrier()
        pltpu.sync_copy(acc.at[pl.ds(sid * rpt, rpt)],
                        out_hbm.at[cid, pl.ds(sid * rpt, rpt)])

    return seg(g, srcp, dstp, zrows)


def _sc_degree(dstp, ones_rows, zrows, nacc, rpt):
    nb = dstp.shape[1]

    @functools.partial(
        pl.kernel,
        mesh=_sc_mesh(),
        compiler_params=pltpu.CompilerParams(use_tc_tiling_on_sc=False),
        out_type=jax.ShapeDtypeStruct((N_SC, nacc, DEG_W), jnp.float32),
        scratch_types=[
            pltpu.VMEM((nb, B_EDGE), jnp.int32),
            pltpu.VMEM((B_EDGE, DEG_W), jnp.float32),
            pltpu.VMEM_SHARED((nacc, DEG_W), jnp.float32),
        ],
    )
    def degk(dst_hbm, ones_hbm, z_hbm, out_hbm, dst_v, ones_v, acc):
        cid = lax.axis_index("c")
        sid = lax.axis_index("s")
        wid = cid * N_TILE + sid

        pltpu.sync_copy(dst_hbm.at[wid], dst_v)
        pltpu.sync_copy(ones_hbm, ones_v)
        pltpu.sync_copy(z_hbm, acc.at[pl.ds(sid * rpt, rpt)])
        plsc.subcore_barrier()

        def body(j, _):
            pltpu.sync_copy(ones_v, acc.at[dst_v.at[j]], add=True)
            return 0

        lax.fori_loop(0, nb, body, 0)
        plsc.subcore_barrier()
        pltpu.sync_copy(acc.at[pl.ds(sid * rpt, rpt)],
                        out_hbm.at[cid, pl.ds(sid * rpt, rpt)])

    return degk(dstp, ones_rows, zrows)


def _tc_matmul(x, w):
    n, k = x.shape
    d = w.shape[1]
    blk = 2000 if n % 2000 == 0 else n
    grid = n // blk

    def body(x_ref, w_ref, h_ref):
        h_ref[...] = jnp.dot(x_ref[...], w_ref[...],
                             preferred_element_type=jnp.float32)

    return pl.pallas_call(
        body,
        grid=(grid,),
        in_specs=[pl.BlockSpec((blk, k), lambda i: (i, 0)),
                  pl.BlockSpec((k, d), lambda i: (0, 0))],
        out_specs=pl.BlockSpec((blk, d), lambda i: (i, 0)),
        out_shape=jax.ShapeDtypeStruct((n, d), jnp.float32),
    )(x, w)


def _tc_dinv_scale(d0, d1, h):
    n, d = h.shape
    blk = 2000 if n % 2000 == 0 else n
    grid = n // blk

    def body(d0_ref, d1_ref, h_ref, dinv_ref, g_ref):
        dinv = lax.rsqrt(d0_ref[...] + d1_ref[...] + 1.0)
        dinv_ref[...] = dinv
        g_ref[...] = dinv * h_ref[...]

    return pl.pallas_call(
        body,
        grid=(grid,),
        in_specs=[pl.BlockSpec((blk, 1), lambda i: (i, 0)),
                  pl.BlockSpec((blk, 1), lambda i: (i, 0)),
                  pl.BlockSpec((blk, d), lambda i: (i, 0))],
        out_specs=[pl.BlockSpec((blk, 1), lambda i: (i, 0)),
                   pl.BlockSpec((blk, d), lambda i: (i, 0))],
        out_shape=[jax.ShapeDtypeStruct((n, 1), jnp.float32),
                   jax.ShapeDtypeStruct((n, d), jnp.float32)],
    )(d0, d1, h)


def _tc_combine_next(s0, s1, gp, dinv, b, w):
    n, d = gp.shape
    dn = w.shape[1]
    blk = 2000 if n % 2000 == 0 else n
    grid = n // blk

    def body(s0_ref, s1_ref, gp_ref, dinv_ref, b_ref, w_ref, g_ref):
        xk = dinv_ref[...] * (s0_ref[...] + s1_ref[...] + gp_ref[...]) + b_ref[...]
        g_ref[...] = dinv_ref[...] * jnp.dot(xk, w_ref[...],
                                             preferred_element_type=jnp.float32)

    return pl.pallas_call(
        body,
        grid=(grid,),
        in_specs=[pl.BlockSpec((blk, d), lambda i: (i, 0)),
                  pl.BlockSpec((blk, d), lambda i: (i, 0)),
                  pl.BlockSpec((blk, d), lambda i: (i, 0)),
                  pl.BlockSpec((blk, 1), lambda i: (i, 0)),
                  pl.BlockSpec((1, d), lambda i: (0, 0)),
                  pl.BlockSpec((d, dn), lambda i: (0, 0))],
        out_specs=pl.BlockSpec((blk, dn), lambda i: (i, 0)),
        out_shape=jax.ShapeDtypeStruct((n, dn), jnp.float32),
    )(s0, s1, gp, dinv, b, w)


def _tc_combine_final(s0, s1, gp, dinv, b):
    n, d = gp.shape
    blk = 2000 if n % 2000 == 0 else n
    grid = n // blk

    def body(s0_ref, s1_ref, gp_ref, dinv_ref, b_ref, o_ref):
        o_ref[...] = dinv_ref[...] * (s0_ref[...] + s1_ref[...] + gp_ref[...]) + b_ref[...]

    return pl.pallas_call(
        body,
        grid=(grid,),
        in_specs=[pl.BlockSpec((blk, d), lambda i: (i, 0)),
                  pl.BlockSpec((blk, d), lambda i: (i, 0)),
                  pl.BlockSpec((blk, d), lambda i: (i, 0)),
                  pl.BlockSpec((blk, 1), lambda i: (i, 0)),
                  pl.BlockSpec((1, d), lambda i: (0, 0))],
        out_specs=pl.BlockSpec((blk, d), lambda i: (i, 0)),
        out_shape=jax.ShapeDtypeStruct((n, d), jnp.float32),
    )(s0, s1, gp, dinv, b)


def kernel(x, edge_index, W1, b1, W2, b2, W3, b3):
    n = x.shape[0]
    e = edge_index.shape[1]

    nb = -(-e // (NW * B_EDGE))
    nb += nb % 2
    ep = NW * nb * B_EDGE
    dummy = n
    rpt = -(-(n + 1) // N_TILE)
    rpt = -(-rpt // 8) * 8
    nacc = N_TILE * rpt

    src = edge_index[0].astype(jnp.int32)
    dst = edge_index[1].astype(jnp.int32)
    srcp = jnp.concatenate([src, jnp.zeros((ep - e,), jnp.int32)]).reshape(NW, nb, B_EDGE)
    dstp = jnp.concatenate([dst, jnp.full((ep - e,), dummy, jnp.int32)]).reshape(NW, nb, B_EDGE)

    ones_rows = jnp.ones((B_EDGE, DEG_W), jnp.float32)
    z_deg = jnp.zeros((rpt, DEG_W), jnp.float32)
    d_hid = W1.shape[1]
    d_out = W3.shape[1]
    dp_hid = -(-d_hid // 16) * 16
    dp_out = -(-d_out // 16) * 16
    z_hid = jnp.zeros((rpt, dp_hid), jnp.float32)
    z_out = jnp.zeros((rpt, dp_out), jnp.float32)

    degp = _sc_degree(dstp, ones_rows, z_deg, nacc, rpt)
    h1 = _tc_matmul(x, W1)
    dinv, g1 = _tc_dinv_scale(degp[0, :n, 0:1], degp[1, :n, 0:1], h1)

    pad_hid = ((0, 0), (0, dp_hid - d_hid))
    pad_out = ((0, 0), (0, dp_out - d_out))

    sp = _sc_segsum(jnp.pad(g1, pad_hid), srcp, dstp, z_hid, nacc, rpt)
    g2 = _tc_combine_next(sp[0, :n, :d_hid], sp[1, :n, :d_hid], g1, dinv,
                          b1.reshape(1, -1), W2)

    sp = _sc_segsum(jnp.pad(g2, pad_hid), srcp, dstp, z_hid, nacc, rpt)
    g3 = _tc_combine_next(sp[0, :n, :d_hid], sp[1, :n, :d_hid], g2, dinv,
                          b2.reshape(1, -1), W3)

    sp = _sc_segsum(jnp.pad(g3, pad_out), srcp, dstp, z_out, nacc, rpt)
    return _tc_combine_final(sp[0, :n, :d_out], sp[1, :n, :d_out], g3, dinv,
                             b3.reshape(1, -1))


# speedup vs baseline: 1.2343x; 1.2343x over previous
import functools

import jax
import jax.numpy as jnp
from jax import lax
from jax.experimental import pallas as pl
from jax.experimental.pallas import tpu as pltpu
from jax.experimental.pallas import tpu_sc as plsc

N_SC = 2
N_TILE = 16
NW = N_SC * N_TILE
B_EDGE = 128
DEG_W = 8


def _sc_mesh():
    return plsc.VectorSubcoreMesh(core_axis_name="c", subcore_axis_name="s")


def _sc_segsum(g, srcp, dstp, zrows, nacc, rpt):
    n_nodes, d = g.shape
    nb = srcp.shape[1]

    @functools.partial(
        pl.kernel,
        mesh=_sc_mesh(),
        compiler_params=pltpu.CompilerParams(use_tc_tiling_on_sc=False),
        out_type=jax.ShapeDtypeStruct((N_SC, nacc, d), jnp.float32),
        scratch_types=[
            pltpu.VMEM((nb, B_EDGE), jnp.int32),
            pltpu.VMEM((nb, B_EDGE), jnp.int32),
            [pltpu.VMEM((B_EDGE, d), jnp.float32)] * 8,
            pltpu.VMEM_SHARED((nacc, d), jnp.float32),
            [pltpu.SemaphoreType.DMA] * 8,
            [pltpu.SemaphoreType.DMA] * 8,
        ],
    )
    def seg(g_hbm, src_hbm, dst_hbm, z_hbm, out_hbm,
            src_v, dst_v, bufs, acc, gsems, ssems):
        cid = lax.axis_index("c")
        sid = lax.axis_index("s")
        wid = cid * N_TILE + sid

        pltpu.sync_copy(src_hbm.at[wid], src_v)
        pltpu.sync_copy(dst_hbm.at[wid], dst_v)
        pltpu.sync_copy(z_hbm, acc.at[pl.ds(sid * rpt, rpt)])
        plsc.subcore_barrier()

        def gather_start(j, b):
            pltpu.make_async_copy(g_hbm.at[src_v.at[j]], bufs[b], gsems[b]).start()

        def gather_wait(b):
            pltpu.make_async_copy(g_hbm.at[src_v.at[0]], bufs[b], gsems[b]).wait()

        def scatter_start(j, b):
            pltpu.make_async_copy(bufs[b], acc.at[dst_v.at[j]], ssems[b]).start(add=True)

        def scatter_wait(b):
            pltpu.make_async_copy(bufs[b], acc.at[dst_v.at[0]], ssems[b]).wait()

        nbuf = 8
        look = nbuf // 2
        for k in range(look):
            gather_start(k, k)

        def body(i, _):
            for b in range(nbuf):
                j = nbuf * i + b
                gather_wait(b)
                scatter_start(j, b)
                nxt = (b + look) % nbuf

                @pl.when(j + look < nb)
                def _():
                    @pl.when(j >= look)
                    def _():
                        scatter_wait(nxt)

                    gather_start(j + look, nxt)
            return 0

        lax.fori_loop(0, nb // nbuf, body, 0)
        for k in range(nbuf):
            scatter_wait(k)
        plsc.subcore_barrier()
        pltpu.sync_copy(acc.at[pl.ds(sid * rpt, rpt)],
                        out_hbm.at[cid, pl.ds(sid * rpt, rpt)])

    return seg(g, srcp, dstp, zrows)


def _sc_degree(dstp, ones_rows, zrows, nacc, rpt):
    nb = dstp.shape[1]

    @functools.partial(
        pl.kernel,
        mesh=_sc_mesh(),
        compiler_params=pltpu.CompilerParams(use_tc_tiling_on_sc=False),
        out_type=jax.ShapeDtypeStruct((N_SC, nacc, DEG_W), jnp.float32),
        scratch_types=[
            pltpu.VMEM((nb, B_EDGE), jnp.int32),
            pltpu.VMEM((B_EDGE, DEG_W), jnp.float32),
            pltpu.VMEM_SHARED((nacc, DEG_W), jnp.float32),
        ],
    )
    def degk(dst_hbm, ones_hbm, z_hbm, out_hbm, dst_v, ones_v, acc):
        cid = lax.axis_index("c")
        sid = lax.axis_index("s")
        wid = cid * N_TILE + sid

        pltpu.sync_copy(dst_hbm.at[wid], dst_v)
        pltpu.sync_copy(ones_hbm, ones_v)
        pltpu.sync_copy(z_hbm, acc.at[pl.ds(sid * rpt, rpt)])
        plsc.subcore_barrier()

        def body(j, _):
            pltpu.sync_copy(ones_v, acc.at[dst_v.at[j]], add=True)
            return 0

        lax.fori_loop(0, nb, body, 0)
        plsc.subcore_barrier()
        pltpu.sync_copy(acc.at[pl.ds(sid * rpt, rpt)],
                        out_hbm.at[cid, pl.ds(sid * rpt, rpt)])

    return degk(dstp, ones_rows, zrows)


def _tc_matmul(x, w):
    n, k = x.shape
    d = w.shape[1]
    blk = 2000 if n % 2000 == 0 else n
    grid = n // blk

    def body(x_ref, w_ref, h_ref):
        h_ref[...] = jnp.dot(x_ref[...], w_ref[...],
                             preferred_element_type=jnp.float32)

    return pl.pallas_call(
        body,
        grid=(grid,),
        in_specs=[pl.BlockSpec((blk, k), lambda i: (i, 0)),
                  pl.BlockSpec((k, d), lambda i: (0, 0))],
        out_specs=pl.BlockSpec((blk, d), lambda i: (i, 0)),
        out_shape=jax.ShapeDtypeStruct((n, d), jnp.float32),
    )(x, w)


def _tc_dinv_scale(d0, d1, h):
    n, d = h.shape
    blk = 2000 if n % 2000 == 0 else n
    grid = n // blk

    def body(d0_ref, d1_ref, h_ref, dinv_ref, g_ref):
        dinv = lax.rsqrt(d0_ref[...] + d1_ref[...] + 1.0)
        dinv_ref[...] = dinv
        g_ref[...] = dinv * h_ref[...]

    return pl.pallas_call(
        body,
        grid=(grid,),
        in_specs=[pl.BlockSpec((blk, 1), lambda i: (i, 0)),
                  pl.BlockSpec((blk, 1), lambda i: (i, 0)),
                  pl.BlockSpec((blk, d), lambda i: (i, 0))],
        out_specs=[pl.BlockSpec((blk, 1), lambda i: (i, 0)),
                   pl.BlockSpec((blk, d), lambda i: (i, 0))],
        out_shape=[jax.ShapeDtypeStruct((n, 1), jnp.float32),
                   jax.ShapeDtypeStruct((n, d), jnp.float32)],
    )(d0, d1, h)


def _tc_combine_next(s0, s1, gp, dinv, b, w):
    n, d = gp.shape
    dn = w.shape[1]
    blk = 2000 if n % 2000 == 0 else n
    grid = n // blk

    def body(s0_ref, s1_ref, gp_ref, dinv_ref, b_ref, w_ref, g_ref):
        xk = dinv_ref[...] * (s0_ref[...] + s1_ref[...] + gp_ref[...]) + b_ref[...]
        g_ref[...] = dinv_ref[...] * jnp.dot(xk, w_ref[...],
                                             preferred_element_type=jnp.float32)

    return pl.pallas_call(
        body,
        grid=(grid,),
        in_specs=[pl.BlockSpec((blk, d), lambda i: (i, 0)),
                  pl.BlockSpec((blk, d), lambda i: (i, 0)),
                  pl.BlockSpec((blk, d), lambda i: (i, 0)),
                  pl.BlockSpec((blk, 1), lambda i: (i, 0)),
                  pl.BlockSpec((1, d), lambda i: (0, 0)),
                  pl.BlockSpec((d, dn), lambda i: (0, 0))],
        out_specs=pl.BlockSpec((blk, dn), lambda i: (i, 0)),
        out_shape=jax.ShapeDtypeStruct((n, dn), jnp.float32),
    )(s0, s1, gp, dinv, b, w)


def _tc_scaled_x(s0, s1, gp, dinv, b):
    n, d = gp.shape
    blk = 2000 if n % 2000 == 0 else n
    grid = n // blk

    def body(s0_ref, s1_ref, gp_ref, dinv_ref, b_ref, y_ref):
        dinv = dinv_ref[...]
        y_ref[...] = dinv * (dinv * (s0_ref[...] + s1_ref[...] + gp_ref[...]) + b_ref[...])

    return pl.pallas_call(
        body,
        grid=(grid,),
        in_specs=[pl.BlockSpec((blk, d), lambda i: (i, 0)),
                  pl.BlockSpec((blk, d), lambda i: (i, 0)),
                  pl.BlockSpec((blk, d), lambda i: (i, 0)),
                  pl.BlockSpec((blk, 1), lambda i: (i, 0)),
                  pl.BlockSpec((1, d), lambda i: (0, 0))],
        out_specs=pl.BlockSpec((blk, d), lambda i: (i, 0)),
        out_shape=jax.ShapeDtypeStruct((n, d), jnp.float32),
    )(s0, s1, gp, dinv, b)


def _tc_combine_final(s0, s1, yp, dinv, b, w):
    n, d = yp.shape
    dn = w.shape[1]
    blk = 2000 if n % 2000 == 0 else n
    grid = n // blk

    def body(s0_ref, s1_ref, yp_ref, dinv_ref, b_ref, w_ref, o_ref):
        u = s0_ref[...] + s1_ref[...] + yp_ref[...]
        o_ref[...] = dinv_ref[...] * jnp.dot(
            u, w_ref[...], preferred_element_type=jnp.float32) + b_ref[...]

    return pl.pallas_call(
        body,
        grid=(grid,),
        in_specs=[pl.BlockSpec((blk, d), lambda i: (i, 0)),
                  pl.BlockSpec((blk, d), lambda i: (i, 0)),
                  pl.BlockSpec((blk, d), lambda i: (i, 0)),
                  pl.BlockSpec((blk, 1), lambda i: (i, 0)),
                  pl.BlockSpec((1, dn), lambda i: (0, 0)),
                  pl.BlockSpec((d, dn), lambda i: (0, 0))],
        out_specs=pl.BlockSpec((blk, dn), lambda i: (i, 0)),
        out_shape=jax.ShapeDtypeStruct((n, dn), jnp.float32),
    )(s0, s1, yp, dinv, b, w)


def kernel(x, edge_index, W1, b1, W2, b2, W3, b3):
    n = x.shape[0]
    e = edge_index.shape[1]

    nb = -(-e // (NW * B_EDGE))
    nb += nb % 2
    ep = NW * nb * B_EDGE
    dummy = n
    rpt = -(-(n + 1) // N_TILE)
    rpt = -(-rpt // 8) * 8
    nacc = N_TILE * rpt

    src = edge_index[0].astype(jnp.int32)
    dst = edge_index[1].astype(jnp.int32)
    srcp = jnp.concatenate([src, jnp.zeros((ep - e,), jnp.int32)]).reshape(NW, nb, B_EDGE)
    dstp = jnp.concatenate([dst, jnp.full((ep - e,), dummy, jnp.int32)]).reshape(NW, nb, B_EDGE)

    ones_rows = jnp.ones((B_EDGE, DEG_W), jnp.float32)
    z_deg = jnp.zeros((rpt, DEG_W), jnp.float32)
    d_hid = W1.shape[1]
    d_out = W3.shape[1]
    dp_hid = -(-d_hid // 16) * 16
    z_hid = jnp.zeros((rpt, dp_hid), jnp.float32)

    degp = _sc_degree(dstp, ones_rows, z_deg, nacc, rpt)
    h1 = _tc_matmul(x, W1)
    dinv, g1 = _tc_dinv_scale(degp[0, :n, 0:1], degp[1, :n, 0:1], h1)

    pad_hid = ((0, 0), (0, dp_hid - d_hid))

    sp = _sc_segsum(jnp.pad(g1, pad_hid), srcp, dstp, z_hid, nacc, rpt)
    g2 = _tc_combine_next(sp[0, :n, :d_hid], sp[1, :n, :d_hid], g1, dinv,
                          b1.reshape(1, -1), W2)

    sp = _sc_segsum(jnp.pad(g2, pad_hid), srcp, dstp, z_hid, nacc, rpt)
    y = _tc_scaled_x(sp[0, :n, :d_hid], sp[1, :n, :d_hid], g2, dinv,
                     b2.reshape(1, -1))

    sp = _sc_segsum(jnp.pad(y, pad_hid), srcp, dstp, z_hid, nacc, rpt)
    return _tc_combine_final(sp[0, :n, :d_hid], sp[1, :n, :d_hid], y, dinv,
                             b3.reshape(1, -1), W3)
